# Initial kernel scaffold; baseline (speedup 1.0000x reference)
import jax
import jax.numpy as jnp
from jax.experimental import pallas as pl


def kernel(x, pe):
    raise NotImplementedError("write your pallas kernel here")


# speedup vs baseline: 4.1000x; 4.1000x over previous
import functools

import jax
import jax.numpy as jnp
from jax import lax
from jax.experimental import pallas as pl
from jax.experimental.pallas import tpu as pltpu
from jax.experimental.pallas import tpu_sc as plsc

D_MODEL = 64
N_IDX = 4096 * 200
LANE = 128
N_ROWS = N_IDX // LANE
NW = 32
ROWS_PER_W = N_ROWS // NW
IDX_BLK = 8
N_BLKS = ROWS_PER_W // IDX_BLK


def _make_gather():
  mesh = plsc.VectorSubcoreMesh(
      core_axis_name="c", subcore_axis_name="s", num_cores=2, num_subcores=16
  )

  @functools.partial(
      pl.kernel,
      mesh=mesh,
      compiler_params=pltpu.CompilerParams(use_tc_tiling_on_sc=False),
      out_type=jax.ShapeDtypeStruct((N_IDX, D_MODEL), jnp.float32),
      scratch_types=[
          pltpu.VMEM((IDX_BLK, LANE), jnp.int32),
          pltpu.VMEM((LANE, D_MODEL), jnp.float32),
          pltpu.SemaphoreType.DMA,
      ],
  )
  def gather_kernel(x_hbm, pe_hbm, out_hbm, idx_v, rows_v, sem):
    wid = lax.axis_index("s") * 2 + lax.axis_index("c")
    row0 = wid * ROWS_PER_W

    def blk(b, carry):
      base_row = row0 + b * IDX_BLK
      pltpu.sync_copy(x_hbm.at[pl.ds(base_row, IDX_BLK)], idx_v)
      for j in range(IDX_BLK):
        pltpu.async_copy(pe_hbm.at[idx_v.at[j]], rows_v, sem).wait()
        pltpu.sync_copy(
            rows_v, out_hbm.at[pl.ds((base_row + j) * LANE, LANE)]
        )
      return carry

    lax.fori_loop(0, N_BLKS, blk, 0)

  return gather_kernel


def kernel(x, pe):
  xf = x.astype(jnp.int32).reshape(N_ROWS, LANE)
  out = _make_gather()(xf, pe)
  return out.reshape(4096, 200, D_MODEL)


# speedup vs baseline: 4.9402x; 1.2049x over previous
import functools

import jax
import jax.numpy as jnp
from jax import lax
from jax.experimental import pallas as pl
from jax.experimental.pallas import tpu as pltpu
from jax.experimental.pallas import tpu_sc as plsc

D_MODEL = 64
N_IDX = 4096 * 200
LANE = 128
N_ROWS = N_IDX // LANE
NW = 32
ROWS_PER_W = N_ROWS // NW
NBUF = 8
N_GRP = ROWS_PER_W // NBUF


def _make_gather():
  mesh = plsc.VectorSubcoreMesh(
      core_axis_name="c", subcore_axis_name="s", num_cores=2, num_subcores=16
  )

  @functools.partial(
      pl.kernel,
      mesh=mesh,
      compiler_params=pltpu.CompilerParams(use_tc_tiling_on_sc=False),
      out_type=jax.ShapeDtypeStruct((N_IDX, D_MODEL), jnp.float32),
      scratch_types=[
          pltpu.VMEM((ROWS_PER_W, LANE), jnp.int32),
          [pltpu.VMEM((LANE, D_MODEL), jnp.float32) for _ in range(NBUF)],
          [pltpu.SemaphoreType.DMA for _ in range(NBUF)],
          [pltpu.SemaphoreType.DMA for _ in range(NBUF)],
      ],
  )
  def gather_kernel(x_hbm, pe_hbm, out_hbm, idx_v, rows, gsem, osem):
    wid = lax.axis_index("s") * 2 + lax.axis_index("c")
    row0 = wid * ROWS_PER_W

    pltpu.sync_copy(x_hbm.at[pl.ds(row0, ROWS_PER_W)], idx_v)

    for b in range(NBUF):
      pltpu.async_copy(pe_hbm.at[idx_v.at[b]], rows[b], gsem[b])

    def grp(gi, carry):
      base = gi * NBUF
      for b in range(NBUF):
        p = base + b
        pltpu.make_async_copy(pe_hbm.at[idx_v.at[p]], rows[b], gsem[b]).wait()
        pltpu.async_copy(
            rows[b], out_hbm.at[pl.ds((row0 + p) * LANE, LANE)], osem[b]
        )

      @pl.when(gi < N_GRP - 1)
      def _():
        for b in range(NBUF):
          p = base + NBUF + b
          pltpu.make_async_copy(
              rows[b], out_hbm.at[pl.ds((row0 + p - NBUF) * LANE, LANE)], osem[b]
          ).wait()
          pltpu.async_copy(pe_hbm.at[idx_v.at[p]], rows[b], gsem[b])

      return carry

    lax.fori_loop(0, N_GRP, grp, 0)

    last = (N_GRP - 1) * NBUF
    for b in range(NBUF):
      pltpu.make_async_copy(
          rows[b], out_hbm.at[pl.ds((row0 + last + b) * LANE, LANE)], osem[b]
      ).wait()

  return gather_kernel


def kernel(x, pe):
  xf = x.astype(jnp.int32).reshape(N_ROWS, LANE)
  out = _make_gather()(xf, pe)
  return out.reshape(4096, 200, D_MODEL)
